# Initial kernel scaffold; baseline (speedup 1.0000x reference)
import jax
import jax.numpy as jnp
from jax.experimental import pallas as pl


def kernel(x, edge_index, W, b, W_phi):
    raise NotImplementedError("write your pallas kernel here")


# speedup vs baseline: 15.1402x; 15.1402x over previous
import functools

import jax
import jax.numpy as jnp
from jax import lax
from jax.experimental import pallas as pl
from jax.experimental.pallas import tpu as pltpu
from jax.experimental.pallas import tpu_sc as plsc

N_NODES = 10000
N_EDGES = 160000
C = 256
HALF = 128
GAMMA = 0.1
EPS = 0.1

NC = 2
NS = 16
CHUNK = 100
E_PER_SUB = N_EDGES // NS
N_CHUNKS = E_PER_SUB // CHUNK
E_PER_WORKER = N_EDGES // (NC * NS)
D_CHUNKS = E_PER_WORKER // CHUNK
N_PAD = 10240
STRIPE = N_PAD // NS

_mesh = plsc.VectorSubcoreMesh(core_axis_name="c", subcore_axis_name="s")


@functools.partial(
    pl.kernel,
    mesh=_mesh,
    out_type=jax.ShapeDtypeStruct((NC, N_PAD, HALF), jnp.float32),
    scratch_types=[
        pltpu.VMEM((D_CHUNKS, CHUNK), jnp.int32),
        pltpu.VMEM((CHUNK, HALF), jnp.float32),
        pltpu.VMEM_SHARED((N_PAD, HALF), jnp.float32),
    ],
)
def _deg_kernel(col_hbm, out_hbm, col_v, ones, acc_sh):
    cid = lax.axis_index("c")
    sid = lax.axis_index("s")
    pltpu.sync_copy(col_hbm.at[cid, sid], col_v)

    @pl.loop(0, CHUNK)
    def _(i):
        @pl.loop(0, HALF // 16)
        def _(k):
            ones[i, pl.ds(k * 16, 16)] = jnp.zeros((16,), jnp.float32)

    @pl.loop(0, STRIPE // CHUNK)
    def _(k):
        pltpu.sync_copy(ones, acc_sh.at[pl.ds(sid * STRIPE + k * CHUNK, CHUNK)])

    pltpu.sync_copy(
        ones.at[pl.ds(0, STRIPE % CHUNK)],
        acc_sh.at[pl.ds(sid * STRIPE + STRIPE - STRIPE % CHUNK, STRIPE % CHUNK)])

    @pl.loop(0, CHUNK)
    def _(i):
        @pl.loop(0, HALF // 16)
        def _(k):
            ones[i, pl.ds(k * 16, 16)] = jnp.ones((16,), jnp.float32)

    plsc.subcore_barrier()

    @pl.loop(0, D_CHUNKS)
    def _(j):
        pltpu.sync_copy(ones, acc_sh.at[col_v.at[j]], add=True)

    plsc.subcore_barrier()
    pltpu.sync_copy(
        acc_sh.at[pl.ds(sid * STRIPE, STRIPE)],
        out_hbm.at[cid, pl.ds(sid * STRIPE, STRIPE)],
    )


@functools.partial(
    pl.kernel,
    mesh=_mesh,
    out_type=jax.ShapeDtypeStruct((NC, N_PAD, HALF), jnp.float32),
    scratch_types=[
        pltpu.VMEM((N_CHUNKS, CHUNK), jnp.int32),
        pltpu.VMEM((N_CHUNKS, CHUNK), jnp.int32),
        pltpu.VMEM((CHUNK, HALF), jnp.float32),
        pltpu.VMEM_SHARED((N_PAD, HALF), jnp.float32),
    ],
)
def _agg_kernel(hs_hbm, row_hbm, col_hbm, out_hbm, row_v, col_v, buf_a, acc_sh):
    cid = lax.axis_index("c")
    sid = lax.axis_index("s")
    pltpu.sync_copy(row_hbm.at[cid, sid], row_v)
    pltpu.sync_copy(col_hbm.at[sid], col_v)

    @pl.loop(0, CHUNK)
    def _(i):
        @pl.loop(0, HALF // 16)
        def _(k):
            buf_a[i, pl.ds(k * 16, 16)] = jnp.zeros((16,), jnp.float32)

    @pl.loop(0, STRIPE // CHUNK)
    def _(k):
        pltpu.sync_copy(buf_a, acc_sh.at[pl.ds(sid * STRIPE + k * CHUNK, CHUNK)])

    pltpu.sync_copy(
        buf_a.at[pl.ds(0, STRIPE % CHUNK)],
        acc_sh.at[pl.ds(sid * STRIPE + STRIPE - STRIPE % CHUNK, STRIPE % CHUNK)])

    plsc.subcore_barrier()

    @pl.loop(0, N_CHUNKS)
    def _(j):
        pltpu.sync_copy(hs_hbm.at[row_v.at[j]], buf_a)
        pltpu.sync_copy(buf_a, acc_sh.at[col_v.at[j]], add=True)

    plsc.subcore_barrier()
    pltpu.sync_copy(
        acc_sh.at[pl.ds(sid * STRIPE, STRIPE)],
        out_hbm.at[cid, pl.ds(sid * STRIPE, STRIPE)],
    )


_BLK = 1000


def _mm_body(x_ref, w_ref, wphi_ref, b_ref, z_ref, h_ref):
    xb = x_ref[...]
    wm = w_ref[...]
    zt = lax.dot_general(xb, wm, (((1,), (1,)), ((), ())),
                         preferred_element_type=jnp.float32)
    z2 = jnp.dot(xb, wm, preferred_element_type=jnp.float32)
    z_ref[...] = zt - z2 - GAMMA * xb + b_ref[...]
    h_ref[...] = jnp.dot(xb, wphi_ref[...], preferred_element_type=jnp.float32)


def _mm_call(x, w, wphi, b2):
    return pl.pallas_call(
        _mm_body,
        grid=(N_NODES // _BLK,),
        in_specs=[
            pl.BlockSpec((_BLK, C), lambda i: (i, 0)),
            pl.BlockSpec((C, C), lambda i: (0, 0)),
            pl.BlockSpec((C, C), lambda i: (0, 0)),
            pl.BlockSpec((1, C), lambda i: (0, 0)),
        ],
        out_specs=[
            pl.BlockSpec((_BLK, C), lambda i: (i, 0)),
            pl.BlockSpec((_BLK, C), lambda i: (i, 0)),
        ],
        out_shape=[
            jax.ShapeDtypeStruct((N_NODES, C), jnp.float32),
            jax.ShapeDtypeStruct((N_NODES, C), jnp.float32),
        ],
    )(x, w, wphi, b2)


def _hs_body(deg_ref, h_ref, hs_ref):
    d = deg_ref[0, :, 0:1] + deg_ref[1, :, 0:1] + 1.0
    dinv = lax.rsqrt(d)
    hb = h_ref[...]
    hs_ref[0] = dinv * hb[:, :HALF]
    hs_ref[1] = dinv * hb[:, HALF:]


def _hs_call(deg2, h):
    return pl.pallas_call(
        _hs_body,
        grid=(N_NODES // _BLK,),
        in_specs=[
            pl.BlockSpec((NC, _BLK, HALF), lambda i: (0, i, 0)),
            pl.BlockSpec((_BLK, C), lambda i: (i, 0)),
        ],
        out_specs=pl.BlockSpec((NC, _BLK, HALF), lambda i: (0, i, 0)),
        out_shape=jax.ShapeDtypeStruct((NC, N_NODES, HALF), jnp.float32),
    )(deg2, h)


def _fin_body(x_ref, z_ref, acc_ref, hs_ref, deg_ref, o_ref):
    d = deg_ref[0, :, 0:1] + deg_ref[1, :, 0:1] + 1.0
    dinv = lax.rsqrt(d)
    accf = jnp.concatenate([acc_ref[0], acc_ref[1]], axis=-1)
    hsf = jnp.concatenate([hs_ref[0], hs_ref[1]], axis=-1)
    conv = z_ref[...] + dinv * (accf + hsf)
    o_ref[...] = x_ref[...] + EPS * jnp.tanh(conv)


def _fin_call(x, z, acc, hs2, deg2):
    return pl.pallas_call(
        _fin_body,
        grid=(N_NODES // _BLK,),
        in_specs=[
            pl.BlockSpec((_BLK, C), lambda i: (i, 0)),
            pl.BlockSpec((_BLK, C), lambda i: (i, 0)),
            pl.BlockSpec((NC, _BLK, HALF), lambda i: (0, i, 0)),
            pl.BlockSpec((NC, _BLK, HALF), lambda i: (0, i, 0)),
            pl.BlockSpec((NC, _BLK, HALF), lambda i: (0, i, 0)),
        ],
        out_specs=pl.BlockSpec((_BLK, C), lambda i: (i, 0)),
        out_shape=jax.ShapeDtypeStruct((N_NODES, C), jnp.float32),
    )(x, z, acc, hs2, deg2)


def kernel(x, edge_index, W, b, W_phi):
    ei = edge_index.astype(jnp.int32)
    row = ei[0]
    col = ei[1]
    col4 = col.reshape(NC, NS, D_CHUNKS, CHUNK)
    col3 = col.reshape(NS, N_CHUNKS, CHUNK)
    row_off = jnp.stack([row, row + N_NODES]).reshape(NC, NS, N_CHUNKS, CHUNK)

    deg2 = _deg_kernel(col4)
    z, h = _mm_call(x, W, W_phi, b.reshape(1, C))
    hs2 = _hs_call(deg2, h)
    acc = _agg_kernel(hs2.reshape(NC * N_NODES, HALF), row_off, col3)
    return _fin_call(x, z, acc, hs2, deg2)


# speedup vs baseline: 17.6843x; 1.1680x over previous
import functools

import jax
import jax.numpy as jnp
from jax import lax
from jax.experimental import pallas as pl
from jax.experimental.pallas import tpu as pltpu
from jax.experimental.pallas import tpu_sc as plsc

N_NODES = 10000
N_EDGES = 160000
C = 256
HALF = 128
GAMMA = 0.1
EPS = 0.1

NC = 2
NS = 16
CHUNK = 100
E_PER_SUB = N_EDGES // NS
N_CHUNKS = E_PER_SUB // CHUNK
E_PER_WORKER = N_EDGES // (NC * NS)
D_CHUNKS = E_PER_WORKER // CHUNK
N_PAD = 10240
STRIPE = N_PAD // NS

_mesh = plsc.VectorSubcoreMesh(core_axis_name="c", subcore_axis_name="s")


@functools.partial(
    pl.kernel,
    mesh=_mesh,
    out_type=jax.ShapeDtypeStruct((NC, N_PAD, HALF), jnp.float32),
    scratch_types=[
        pltpu.VMEM((D_CHUNKS, CHUNK), jnp.int32),
        pltpu.VMEM((CHUNK, HALF), jnp.float32),
        pltpu.VMEM_SHARED((N_PAD, HALF), jnp.float32),
        pltpu.SemaphoreType.DMA,
    ],
)
def _deg_kernel(col_hbm, out_hbm, col_v, ones, acc_sh, sem):
    cid = lax.axis_index("c")
    sid = lax.axis_index("s")
    pltpu.sync_copy(col_hbm.at[cid, sid], col_v)

    @pl.loop(0, CHUNK)
    def _(i):
        @pl.loop(0, HALF // 16)
        def _(k):
            ones[i, pl.ds(k * 16, 16)] = jnp.zeros((16,), jnp.float32)

    @pl.loop(0, STRIPE // CHUNK)
    def _(k):
        pltpu.sync_copy(ones, acc_sh.at[pl.ds(sid * STRIPE + k * CHUNK, CHUNK)])

    pltpu.sync_copy(
        ones.at[pl.ds(0, STRIPE % CHUNK)],
        acc_sh.at[pl.ds(sid * STRIPE + STRIPE - STRIPE % CHUNK, STRIPE % CHUNK)])

    @pl.loop(0, CHUNK)
    def _(i):
        @pl.loop(0, HALF // 16)
        def _(k):
            ones[i, pl.ds(k * 16, 16)] = jnp.ones((16,), jnp.float32)

    plsc.subcore_barrier()

    @pl.loop(0, D_CHUNKS)
    def _(j):
        pltpu.async_copy(ones, acc_sh.at[col_v.at[j]], sem, add=True)

    @pl.loop(0, D_CHUNKS)
    def _(j):
        pltpu.make_async_copy(ones, acc_sh.at[col_v.at[0]], sem).wait()

    plsc.subcore_barrier()
    pltpu.sync_copy(
        acc_sh.at[pl.ds(sid * STRIPE, STRIPE)],
        out_hbm.at[cid, pl.ds(sid * STRIPE, STRIPE)],
    )


@functools.partial(
    pl.kernel,
    mesh=_mesh,
    out_type=jax.ShapeDtypeStruct((NC, N_PAD, HALF), jnp.float32),
    scratch_types=[
        pltpu.VMEM((N_CHUNKS, CHUNK), jnp.int32),
        pltpu.VMEM((2, CHUNK), jnp.int32),
        pltpu.VMEM((CHUNK, HALF), jnp.float32),
        pltpu.VMEM((CHUNK, HALF), jnp.float32),
        pltpu.VMEM_SHARED((N_PAD, HALF), jnp.float32),
        pltpu.SemaphoreType.DMA,
        pltpu.SemaphoreType.DMA,
        pltpu.SemaphoreType.DMA,
        pltpu.SemaphoreType.DMA,
        pltpu.SemaphoreType.DMA,
        pltpu.SemaphoreType.DMA,
    ],
)
def _agg_kernel(hs_hbm, row_hbm, col_hbm, out_hbm, row_v, col_v, buf_a, buf_b,
                acc_sh, sem_ga, sem_gb, sem_ca, sem_cb, sem_sa, sem_sb):
    cid = lax.axis_index("c")
    sid = lax.axis_index("s")
    pltpu.sync_copy(row_hbm.at[cid, sid], row_v)

    @pl.loop(0, CHUNK)
    def _(i):
        @pl.loop(0, HALF // 16)
        def _(k):
            buf_a[i, pl.ds(k * 16, 16)] = jnp.zeros((16,), jnp.float32)

    @pl.loop(0, STRIPE // CHUNK)
    def _(k):
        pltpu.sync_copy(buf_a, acc_sh.at[pl.ds(sid * STRIPE + k * CHUNK, CHUNK)])

    pltpu.sync_copy(
        buf_a.at[pl.ds(0, STRIPE % CHUNK)],
        acc_sh.at[pl.ds(sid * STRIPE + STRIPE - STRIPE % CHUNK, STRIPE % CHUNK)])

    plsc.subcore_barrier()

    pltpu.async_copy(col_hbm.at[sid, 0], col_v.at[0], sem_ca)
    pltpu.async_copy(col_hbm.at[sid, 1], col_v.at[1], sem_cb)
    pltpu.async_copy(hs_hbm.at[row_v.at[0]], buf_a, sem_ga)
    pltpu.async_copy(hs_hbm.at[row_v.at[1]], buf_b, sem_gb)

    @pl.loop(0, N_CHUNKS, step=2)
    def _(j):
        ja = jnp.minimum(j + 2, N_CHUNKS - 1)
        jb = jnp.minimum(j + 3, N_CHUNKS - 1)
        pltpu.make_async_copy(hs_hbm.at[row_v.at[0]], buf_a, sem_ga).wait()
        pltpu.make_async_copy(col_hbm.at[sid, 0], col_v.at[0], sem_ca).wait()
        pltpu.async_copy(buf_a, acc_sh.at[col_v.at[0]], sem_sa, add=True)
        pltpu.make_async_copy(hs_hbm.at[row_v.at[0]], buf_b, sem_gb).wait()
        pltpu.make_async_copy(col_hbm.at[sid, 0], col_v.at[1], sem_cb).wait()
        pltpu.async_copy(buf_b, acc_sh.at[col_v.at[1]], sem_sb, add=True)
        pltpu.make_async_copy(buf_a, acc_sh.at[col_v.at[0]], sem_sa).wait()
        pltpu.async_copy(col_hbm.at[sid, ja], col_v.at[0], sem_ca)
        pltpu.async_copy(hs_hbm.at[row_v.at[ja]], buf_a, sem_ga)
        pltpu.make_async_copy(buf_b, acc_sh.at[col_v.at[1]], sem_sb).wait()
        pltpu.async_copy(col_hbm.at[sid, jb], col_v.at[1], sem_cb)
        pltpu.async_copy(hs_hbm.at[row_v.at[jb]], buf_b, sem_gb)

    pltpu.make_async_copy(hs_hbm.at[row_v.at[0]], buf_a, sem_ga).wait()
    pltpu.make_async_copy(hs_hbm.at[row_v.at[0]], buf_b, sem_gb).wait()
    pltpu.make_async_copy(col_hbm.at[sid, 0], col_v.at[0], sem_ca).wait()
    pltpu.make_async_copy(col_hbm.at[sid, 0], col_v.at[1], sem_cb).wait()

    plsc.subcore_barrier()
    pltpu.sync_copy(
        acc_sh.at[pl.ds(sid * STRIPE, STRIPE)],
        out_hbm.at[cid, pl.ds(sid * STRIPE, STRIPE)],
    )


_BLK = 1000


def _mm_body(x_ref, w_ref, wphi_ref, b_ref, z_ref, h_ref):
    xb = x_ref[...]
    wm = w_ref[...]
    zt = lax.dot_general(xb, wm, (((1,), (1,)), ((), ())),
                         preferred_element_type=jnp.float32)
    z2 = jnp.dot(xb, wm, preferred_element_type=jnp.float32)
    z_ref[...] = zt - z2 - GAMMA * xb + b_ref[...]
    h_ref[...] = jnp.dot(xb, wphi_ref[...], preferred_element_type=jnp.float32)


def _mm_call(x, w, wphi, b2):
    return pl.pallas_call(
        _mm_body,
        grid=(N_NODES // _BLK,),
        in_specs=[
            pl.BlockSpec((_BLK, C), lambda i: (i, 0)),
            pl.BlockSpec((C, C), lambda i: (0, 0)),
            pl.BlockSpec((C, C), lambda i: (0, 0)),
            pl.BlockSpec((1, C), lambda i: (0, 0)),
        ],
        out_specs=[
            pl.BlockSpec((_BLK, C), lambda i: (i, 0)),
            pl.BlockSpec((_BLK, C), lambda i: (i, 0)),
        ],
        out_shape=[
            jax.ShapeDtypeStruct((N_NODES, C), jnp.float32),
            jax.ShapeDtypeStruct((N_NODES, C), jnp.float32),
        ],
    )(x, w, wphi, b2)


def _hs_body(deg_ref, h_ref, hs_ref):
    d = deg_ref[0, :, 0:1] + deg_ref[1, :, 0:1] + 1.0
    dinv = lax.rsqrt(d)
    hb = h_ref[...]
    hs_ref[0] = dinv * hb[:, :HALF]
    hs_ref[1] = dinv * hb[:, HALF:]


def _hs_call(deg2, h):
    return pl.pallas_call(
        _hs_body,
        grid=(N_NODES // _BLK,),
        in_specs=[
            pl.BlockSpec((NC, _BLK, HALF), lambda i: (0, i, 0)),
            pl.BlockSpec((_BLK, C), lambda i: (i, 0)),
        ],
        out_specs=pl.BlockSpec((NC, _BLK, HALF), lambda i: (0, i, 0)),
        out_shape=jax.ShapeDtypeStruct((NC, N_NODES, HALF), jnp.float32),
    )(deg2, h)


def _fin_body(x_ref, z_ref, acc_ref, hs_ref, deg_ref, o_ref):
    d = deg_ref[0, :, 0:1] + deg_ref[1, :, 0:1] + 1.0
    dinv = lax.rsqrt(d)
    accf = jnp.concatenate([acc_ref[0], acc_ref[1]], axis=-1)
    hsf = jnp.concatenate([hs_ref[0], hs_ref[1]], axis=-1)
    conv = z_ref[...] + dinv * (accf + hsf)
    o_ref[...] = x_ref[...] + EPS * jnp.tanh(conv)


def _fin_call(x, z, acc, hs2, deg2):
    return pl.pallas_call(
        _fin_body,
        grid=(N_NODES // _BLK,),
        in_specs=[
            pl.BlockSpec((_BLK, C), lambda i: (i, 0)),
            pl.BlockSpec((_BLK, C), lambda i: (i, 0)),
            pl.BlockSpec((NC, _BLK, HALF), lambda i: (0, i, 0)),
            pl.BlockSpec((NC, _BLK, HALF), lambda i: (0, i, 0)),
            pl.BlockSpec((NC, _BLK, HALF), lambda i: (0, i, 0)),
        ],
        out_specs=pl.BlockSpec((_BLK, C), lambda i: (i, 0)),
        out_shape=jax.ShapeDtypeStruct((N_NODES, C), jnp.float32),
    )(x, z, acc, hs2, deg2)


def kernel(x, edge_index, W, b, W_phi):
    ei = edge_index.astype(jnp.int32)
    row = ei[0]
    col = ei[1]
    col4 = col.reshape(NC, NS, D_CHUNKS, CHUNK)
    col3 = col.reshape(NS, N_CHUNKS, CHUNK)
    row_off = jnp.stack([row, row + N_NODES]).reshape(NC, NS, N_CHUNKS, CHUNK)

    deg2 = _deg_kernel(col4)
    z, h = _mm_call(x, W, W_phi, b.reshape(1, C))
    hs2 = _hs_call(deg2, h)
    acc = _agg_kernel(hs2.reshape(NC * N_NODES, HALF), row_off, col3)
    return _fin_call(x, z, acc, hs2, deg2)


# speedup vs baseline: 18.8660x; 1.0668x over previous
import dataclasses
import functools

import jax
import jax.numpy as jnp
from jax import lax
from jax.experimental import pallas as pl
from jax.experimental.pallas import tpu as pltpu
from jax.experimental.pallas import tpu_sc as plsc

N_NODES = 10000
N_EDGES = 160000
C = 256
HALF = 128
GAMMA = 0.1
EPS = 0.1

NC = 2
NS = 16
CHUNK = 100
E_PER_SUB = N_EDGES // NS
N_CHUNKS = E_PER_SUB // CHUNK
E_PER_WORKER = N_EDGES // (NC * NS)
D_CHUNKS = E_PER_WORKER // CHUNK
N_PAD = 10240
STRIPE = N_PAD // NS

_mesh = plsc.VectorSubcoreMesh(core_axis_name="c", subcore_axis_name="s")


E_PAD_W = 5120
DV_CHUNKS = E_PAD_W // 16

_cp = pltpu.CompilerParams()
if "needs_layout_passes" in pltpu.CompilerParams.__dataclass_fields__:
    _cp = dataclasses.replace(_cp, needs_layout_passes=False)


@functools.partial(
    pl.kernel,
    mesh=_mesh,
    compiler_params=_cp,
    out_type=jax.ShapeDtypeStruct((NC, N_PAD), jnp.float32),
    scratch_types=[
        pltpu.VMEM((DV_CHUNKS, 16), jnp.int32),
        pltpu.VMEM((N_PAD,), jnp.float32),
        pltpu.VMEM((STRIPE,), jnp.float32),
        pltpu.VMEM((STRIPE,), jnp.float32),
        pltpu.VMEM_SHARED((NS, N_PAD), jnp.float32),
    ],
)
def _deg_kernel(col_hbm, out_hbm, col_v, hist, tmp, accs, stage_sh):
    cid = lax.axis_index("c")
    sid = lax.axis_index("s")
    pltpu.sync_copy(col_hbm.at[cid, sid], col_v)

    @pl.loop(0, N_PAD // 16)
    def _(i):
        hist[pl.ds(i * 16, 16)] = jnp.zeros((16,), jnp.float32)

    one16 = jnp.ones((16,), jnp.float32)

    @pl.loop(0, DV_CHUNKS)
    def _(i):
        plsc.addupdate_scatter(hist, [col_v[i, :]], one16)

    pltpu.sync_copy(hist, stage_sh.at[sid])
    plsc.subcore_barrier()

    @pl.loop(0, STRIPE // 16)
    def _(t):
        accs[pl.ds(t * 16, 16)] = jnp.zeros((16,), jnp.float32)

    @pl.loop(0, NS)
    def _(k):
        pltpu.sync_copy(stage_sh.at[k, pl.ds(sid * STRIPE, STRIPE)], tmp)

        @pl.loop(0, STRIPE // 16)
        def _(t):
            sl = pl.ds(t * 16, 16)
            accs[sl] = accs[sl] + tmp[sl]

    pltpu.sync_copy(accs, out_hbm.at[cid, pl.ds(sid * STRIPE, STRIPE)])


@functools.partial(
    pl.kernel,
    mesh=_mesh,
    out_type=jax.ShapeDtypeStruct((NC, N_PAD, HALF), jnp.float32),
    scratch_types=[
        pltpu.VMEM((N_CHUNKS, CHUNK), jnp.int32),
        pltpu.VMEM((2, CHUNK), jnp.int32),
        pltpu.VMEM((CHUNK, HALF), jnp.float32),
        pltpu.VMEM((CHUNK, HALF), jnp.float32),
        pltpu.VMEM_SHARED((N_PAD, HALF), jnp.float32),
        pltpu.SemaphoreType.DMA,
        pltpu.SemaphoreType.DMA,
        pltpu.SemaphoreType.DMA,
        pltpu.SemaphoreType.DMA,
        pltpu.SemaphoreType.DMA,
        pltpu.SemaphoreType.DMA,
    ],
)
def _agg_kernel(hs_hbm, row_hbm, col_hbm, out_hbm, row_v, col_v, buf_a, buf_b,
                acc_sh, sem_ga, sem_gb, sem_ca, sem_cb, sem_sa, sem_sb):
    cid = lax.axis_index("c")
    sid = lax.axis_index("s")
    pltpu.sync_copy(row_hbm.at[cid, sid], row_v)

    @pl.loop(0, CHUNK)
    def _(i):
        @pl.loop(0, HALF // 16)
        def _(k):
            buf_a[i, pl.ds(k * 16, 16)] = jnp.zeros((16,), jnp.float32)

    @pl.loop(0, STRIPE // CHUNK)
    def _(k):
        pltpu.sync_copy(buf_a, acc_sh.at[pl.ds(sid * STRIPE + k * CHUNK, CHUNK)])

    pltpu.sync_copy(
        buf_a.at[pl.ds(0, STRIPE % CHUNK)],
        acc_sh.at[pl.ds(sid * STRIPE + STRIPE - STRIPE % CHUNK, STRIPE % CHUNK)])

    plsc.subcore_barrier()

    pltpu.async_copy(col_hbm.at[sid, 0], col_v.at[0], sem_ca)
    pltpu.async_copy(col_hbm.at[sid, 1], col_v.at[1], sem_cb)
    pltpu.async_copy(hs_hbm.at[row_v.at[0]], buf_a, sem_ga)
    pltpu.async_copy(hs_hbm.at[row_v.at[1]], buf_b, sem_gb)

    @pl.loop(0, N_CHUNKS, step=2)
    def _(j):
        ja = jnp.minimum(j + 2, N_CHUNKS - 1)
        jb = jnp.minimum(j + 3, N_CHUNKS - 1)
        pltpu.make_async_copy(hs_hbm.at[row_v.at[0]], buf_a, sem_ga).wait()
        pltpu.make_async_copy(col_hbm.at[sid, 0], col_v.at[0], sem_ca).wait()
        pltpu.async_copy(buf_a, acc_sh.at[col_v.at[0]], sem_sa, add=True)
        pltpu.make_async_copy(hs_hbm.at[row_v.at[0]], buf_b, sem_gb).wait()
        pltpu.make_async_copy(col_hbm.at[sid, 0], col_v.at[1], sem_cb).wait()
        pltpu.async_copy(buf_b, acc_sh.at[col_v.at[1]], sem_sb, add=True)
        pltpu.make_async_copy(buf_a, acc_sh.at[col_v.at[0]], sem_sa).wait()
        pltpu.async_copy(col_hbm.at[sid, ja], col_v.at[0], sem_ca)
        pltpu.async_copy(hs_hbm.at[row_v.at[ja]], buf_a, sem_ga)
        pltpu.make_async_copy(buf_b, acc_sh.at[col_v.at[1]], sem_sb).wait()
        pltpu.async_copy(col_hbm.at[sid, jb], col_v.at[1], sem_cb)
        pltpu.async_copy(hs_hbm.at[row_v.at[jb]], buf_b, sem_gb)

    pltpu.make_async_copy(hs_hbm.at[row_v.at[0]], buf_a, sem_ga).wait()
    pltpu.make_async_copy(hs_hbm.at[row_v.at[0]], buf_b, sem_gb).wait()
    pltpu.make_async_copy(col_hbm.at[sid, 0], col_v.at[0], sem_ca).wait()
    pltpu.make_async_copy(col_hbm.at[sid, 0], col_v.at[1], sem_cb).wait()

    plsc.subcore_barrier()
    pltpu.sync_copy(
        acc_sh.at[pl.ds(sid * STRIPE, STRIPE)],
        out_hbm.at[cid, pl.ds(sid * STRIPE, STRIPE)],
    )


_BLK = 1000


def _mm_body(x_ref, w_ref, wphi_ref, b_ref, z_ref, h_ref):
    xb = x_ref[...]
    wm = w_ref[...]
    zt = lax.dot_general(xb, wm, (((1,), (1,)), ((), ())),
                         preferred_element_type=jnp.float32)
    z2 = jnp.dot(xb, wm, preferred_element_type=jnp.float32)
    z_ref[...] = zt - z2 - GAMMA * xb + b_ref[...]
    h_ref[...] = jnp.dot(xb, wphi_ref[...], preferred_element_type=jnp.float32)


def _mm_call(x, w, wphi, b2):
    return pl.pallas_call(
        _mm_body,
        grid=(N_NODES // _BLK,),
        in_specs=[
            pl.BlockSpec((_BLK, C), lambda i: (i, 0)),
            pl.BlockSpec((C, C), lambda i: (0, 0)),
            pl.BlockSpec((C, C), lambda i: (0, 0)),
            pl.BlockSpec((1, C), lambda i: (0, 0)),
        ],
        out_specs=[
            pl.BlockSpec((_BLK, C), lambda i: (i, 0)),
            pl.BlockSpec((_BLK, C), lambda i: (i, 0)),
        ],
        out_shape=[
            jax.ShapeDtypeStruct((N_NODES, C), jnp.float32),
            jax.ShapeDtypeStruct((N_NODES, C), jnp.float32),
        ],
    )(x, w, wphi, b2)


def _hs_body(deg_ref, h_ref, hs_ref):
    d = deg_ref[:, 0:1] + deg_ref[:, 1:2] + 1.0
    dinv = lax.rsqrt(d)
    hb = h_ref[...]
    hs_ref[0] = dinv * hb[:, :HALF]
    hs_ref[1] = dinv * hb[:, HALF:]


def _hs_call(deg2, h):
    return pl.pallas_call(
        _hs_body,
        grid=(N_NODES // _BLK,),
        in_specs=[
            pl.BlockSpec((_BLK, NC), lambda i: (i, 0)),
            pl.BlockSpec((_BLK, C), lambda i: (i, 0)),
        ],
        out_specs=pl.BlockSpec((NC, _BLK, HALF), lambda i: (0, i, 0)),
        out_shape=jax.ShapeDtypeStruct((NC, N_NODES, HALF), jnp.float32),
    )(deg2, h)


def _fin_body(x_ref, z_ref, acc_ref, hs_ref, deg_ref, o_ref):
    d = deg_ref[:, 0:1] + deg_ref[:, 1:2] + 1.0
    dinv = lax.rsqrt(d)
    accf = jnp.concatenate([acc_ref[0], acc_ref[1]], axis=-1)
    hsf = jnp.concatenate([hs_ref[0], hs_ref[1]], axis=-1)
    conv = z_ref[...] + dinv * (accf + hsf)
    o_ref[...] = x_ref[...] + EPS * jnp.tanh(conv)


def _fin_call(x, z, acc, hs2, deg2):
    return pl.pallas_call(
        _fin_body,
        grid=(N_NODES // _BLK,),
        in_specs=[
            pl.BlockSpec((_BLK, C), lambda i: (i, 0)),
            pl.BlockSpec((_BLK, C), lambda i: (i, 0)),
            pl.BlockSpec((NC, _BLK, HALF), lambda i: (0, i, 0)),
            pl.BlockSpec((NC, _BLK, HALF), lambda i: (0, i, 0)),
            pl.BlockSpec((_BLK, NC), lambda i: (i, 0)),
        ],
        out_specs=pl.BlockSpec((_BLK, C), lambda i: (i, 0)),
        out_shape=jax.ShapeDtypeStruct((N_NODES, C), jnp.float32),
    )(x, z, acc, hs2, deg2)


def kernel(x, edge_index, W, b, W_phi):
    ei = edge_index.astype(jnp.int32)
    row = ei[0]
    col = ei[1]
    col5 = jnp.concatenate(
        [col.reshape(NC * NS, E_PER_WORKER),
         jnp.full((NC * NS, E_PAD_W - E_PER_WORKER), N_NODES + 8, jnp.int32)],
        axis=1).reshape(NC, NS, DV_CHUNKS, 16)
    col3 = col.reshape(NS, N_CHUNKS, CHUNK)
    row_off = jnp.stack([row, row + N_NODES]).reshape(NC, NS, N_CHUNKS, CHUNK)

    deg2 = _deg_kernel(col5).T
    z, h = _mm_call(x, W, W_phi, b.reshape(1, C))
    hs2 = _hs_call(deg2, h)
    acc = _agg_kernel(hs2.reshape(NC * N_NODES, HALF), row_off, col3)
    return _fin_call(x, z, acc, hs2, deg2)


# speedup vs baseline: 19.7385x; 1.0462x over previous
import dataclasses
import functools

import jax
import jax.numpy as jnp
from jax import lax
from jax.experimental import pallas as pl
from jax.experimental.pallas import tpu as pltpu
from jax.experimental.pallas import tpu_sc as plsc

N_NODES = 10000
N_EDGES = 160000
C = 256
HALF = 128
GAMMA = 0.1
EPS = 0.1

NC = 2
NS = 16
CHUNK = 125
E_PER_SUB = N_EDGES // NS
N_CHUNKS = E_PER_SUB // CHUNK
E_PER_WORKER = N_EDGES // (NC * NS)
N_PAD = 10240
STRIPE = N_PAD // NS

_mesh = plsc.VectorSubcoreMesh(core_axis_name="c", subcore_axis_name="s")


E_PAD_W = 5120
DV_CHUNKS = E_PAD_W // 16

_cp = pltpu.CompilerParams()
if "needs_layout_passes" in pltpu.CompilerParams.__dataclass_fields__:
    _cp = dataclasses.replace(_cp, needs_layout_passes=False)


@functools.partial(
    pl.kernel,
    mesh=_mesh,
    compiler_params=_cp,
    out_type=jax.ShapeDtypeStruct((NC, N_PAD), jnp.float32),
    scratch_types=[
        pltpu.VMEM((DV_CHUNKS, 16), jnp.int32),
        pltpu.VMEM((N_PAD,), jnp.float32),
        pltpu.VMEM((STRIPE,), jnp.float32),
        pltpu.VMEM((STRIPE,), jnp.float32),
        pltpu.VMEM_SHARED((NS, N_PAD), jnp.float32),
    ],
)
def _deg_kernel(col_hbm, out_hbm, col_v, hist, tmp, accs, stage_sh):
    cid = lax.axis_index("c")
    sid = lax.axis_index("s")
    pltpu.sync_copy(col_hbm.at[cid, sid], col_v)

    @pl.loop(0, N_PAD // 16)
    def _(i):
        hist[pl.ds(i * 16, 16)] = jnp.zeros((16,), jnp.float32)

    one16 = jnp.ones((16,), jnp.float32)

    @pl.loop(0, DV_CHUNKS)
    def _(i):
        plsc.addupdate_scatter(hist, [col_v[i, :]], one16)

    pltpu.sync_copy(hist, stage_sh.at[sid])
    plsc.subcore_barrier()

    @pl.loop(0, STRIPE // 16)
    def _(t):
        accs[pl.ds(t * 16, 16)] = jnp.zeros((16,), jnp.float32)

    @pl.loop(0, NS)
    def _(k):
        pltpu.sync_copy(stage_sh.at[k, pl.ds(sid * STRIPE, STRIPE)], tmp)

        @pl.loop(0, STRIPE // 16)
        def _(t):
            sl = pl.ds(t * 16, 16)
            accs[sl] = accs[sl] + tmp[sl]

    pltpu.sync_copy(accs, out_hbm.at[cid, pl.ds(sid * STRIPE, STRIPE)])


@functools.partial(
    pl.kernel,
    mesh=_mesh,
    out_type=jax.ShapeDtypeStruct((NC, N_PAD, HALF), jnp.float32),
    scratch_types=[
        pltpu.VMEM((N_CHUNKS, CHUNK), jnp.int32),
        pltpu.VMEM((2, CHUNK), jnp.int32),
        pltpu.VMEM((CHUNK, HALF), jnp.float32),
        pltpu.VMEM((CHUNK, HALF), jnp.float32),
        pltpu.VMEM_SHARED((N_PAD, HALF), jnp.float32),
        pltpu.SemaphoreType.DMA,
        pltpu.SemaphoreType.DMA,
        pltpu.SemaphoreType.DMA,
        pltpu.SemaphoreType.DMA,
        pltpu.SemaphoreType.DMA,
        pltpu.SemaphoreType.DMA,
    ],
)
def _agg_kernel(hs_hbm, row_hbm, col_hbm, out_hbm, row_v, col_v, buf_a, buf_b,
                acc_sh, sem_ga, sem_gb, sem_ca, sem_cb, sem_sa, sem_sb):
    cid = lax.axis_index("c")
    sid = lax.axis_index("s")
    pltpu.sync_copy(row_hbm.at[cid, sid], row_v)

    @pl.when(sid < NS - 1)
    def _():
        pltpu.sync_copy(
            hs_hbm.at[pl.ds(cid * N_NODES + sid * STRIPE, STRIPE)],
            acc_sh.at[pl.ds(sid * STRIPE, STRIPE)])

    @pl.when(sid == NS - 1)
    def _():
        pltpu.sync_copy(
            hs_hbm.at[pl.ds(cid * N_NODES + (NS - 1) * STRIPE,
                            N_NODES - (NS - 1) * STRIPE)],
            acc_sh.at[pl.ds((NS - 1) * STRIPE, N_NODES - (NS - 1) * STRIPE)])

    plsc.subcore_barrier()

    pltpu.async_copy(col_hbm.at[sid, 0], col_v.at[0], sem_ca)
    pltpu.async_copy(col_hbm.at[sid, 1], col_v.at[1], sem_cb)
    pltpu.async_copy(hs_hbm.at[row_v.at[0]], buf_a, sem_ga)
    pltpu.async_copy(hs_hbm.at[row_v.at[1]], buf_b, sem_gb)

    @pl.loop(0, N_CHUNKS, step=2)
    def _(j):
        ja = jnp.minimum(j + 2, N_CHUNKS - 1)
        jb = jnp.minimum(j + 3, N_CHUNKS - 1)
        pltpu.make_async_copy(hs_hbm.at[row_v.at[0]], buf_a, sem_ga).wait()
        pltpu.make_async_copy(col_hbm.at[sid, 0], col_v.at[0], sem_ca).wait()
        pltpu.async_copy(buf_a, acc_sh.at[col_v.at[0]], sem_sa, add=True)
        pltpu.make_async_copy(hs_hbm.at[row_v.at[0]], buf_b, sem_gb).wait()
        pltpu.make_async_copy(col_hbm.at[sid, 0], col_v.at[1], sem_cb).wait()
        pltpu.async_copy(buf_b, acc_sh.at[col_v.at[1]], sem_sb, add=True)
        pltpu.make_async_copy(buf_a, acc_sh.at[col_v.at[0]], sem_sa).wait()
        pltpu.async_copy(col_hbm.at[sid, ja], col_v.at[0], sem_ca)
        pltpu.async_copy(hs_hbm.at[row_v.at[ja]], buf_a, sem_ga)
        pltpu.make_async_copy(buf_b, acc_sh.at[col_v.at[1]], sem_sb).wait()
        pltpu.async_copy(col_hbm.at[sid, jb], col_v.at[1], sem_cb)
        pltpu.async_copy(hs_hbm.at[row_v.at[jb]], buf_b, sem_gb)

    pltpu.make_async_copy(hs_hbm.at[row_v.at[0]], buf_a, sem_ga).wait()
    pltpu.make_async_copy(hs_hbm.at[row_v.at[0]], buf_b, sem_gb).wait()
    pltpu.make_async_copy(col_hbm.at[sid, 0], col_v.at[0], sem_ca).wait()
    pltpu.make_async_copy(col_hbm.at[sid, 0], col_v.at[1], sem_cb).wait()

    plsc.subcore_barrier()
    pltpu.sync_copy(
        acc_sh.at[pl.ds(sid * STRIPE, STRIPE)],
        out_hbm.at[cid, pl.ds(sid * STRIPE, STRIPE)],
    )


_BLK = 1000


def _mmhs_body(x_ref, wphi_ref, deg_ref, hs_ref):
    d = deg_ref[:, 0:1] + deg_ref[:, 1:2] + 1.0
    dinv = lax.rsqrt(d)
    h = jnp.dot(x_ref[...], wphi_ref[...], preferred_element_type=jnp.float32)
    hs_ref[0] = dinv * h[:, :HALF]
    hs_ref[1] = dinv * h[:, HALF:]


def _mmhs_call(x, wphi, deg2):
    return pl.pallas_call(
        _mmhs_body,
        grid=(N_NODES // _BLK,),
        in_specs=[
            pl.BlockSpec((_BLK, C), lambda i: (i, 0)),
            pl.BlockSpec((C, C), lambda i: (0, 0)),
            pl.BlockSpec((_BLK, NC), lambda i: (i, 0)),
        ],
        out_specs=pl.BlockSpec((NC, _BLK, HALF), lambda i: (0, i, 0)),
        out_shape=jax.ShapeDtypeStruct((NC, N_NODES, HALF), jnp.float32),
    )(x, wphi, deg2)


def _fin_body(x_ref, w_ref, b_ref, acc_ref, deg_ref, o_ref):
    xb = x_ref[...]
    wm = w_ref[...]
    d = deg_ref[:, 0:1] + deg_ref[:, 1:2] + 1.0
    dinv = lax.rsqrt(d)
    zt = lax.dot_general(xb, wm, (((1,), (1,)), ((), ())),
                         preferred_element_type=jnp.float32)
    z2 = jnp.dot(xb, wm, preferred_element_type=jnp.float32)
    z = zt - z2 - GAMMA * xb + b_ref[...]
    accf = jnp.concatenate([acc_ref[0], acc_ref[1]], axis=-1)
    o_ref[...] = xb + EPS * jnp.tanh(z + dinv * accf)


def _fin_call(x, w, b2, acc, deg2):
    return pl.pallas_call(
        _fin_body,
        grid=(N_NODES // _BLK,),
        in_specs=[
            pl.BlockSpec((_BLK, C), lambda i: (i, 0)),
            pl.BlockSpec((C, C), lambda i: (0, 0)),
            pl.BlockSpec((1, C), lambda i: (0, 0)),
            pl.BlockSpec((NC, _BLK, HALF), lambda i: (0, i, 0)),
            pl.BlockSpec((_BLK, NC), lambda i: (i, 0)),
        ],
        out_specs=pl.BlockSpec((_BLK, C), lambda i: (i, 0)),
        out_shape=jax.ShapeDtypeStruct((N_NODES, C), jnp.float32),
    )(x, w, b2, acc, deg2)


def kernel(x, edge_index, W, b, W_phi):
    ei = edge_index.astype(jnp.int32)
    row = ei[0]
    col = ei[1]
    col5 = jnp.concatenate(
        [col.reshape(NC * NS, E_PER_WORKER),
         jnp.full((NC * NS, E_PAD_W - E_PER_WORKER), N_NODES + 8, jnp.int32)],
        axis=1).reshape(NC, NS, DV_CHUNKS, 16)
    col3 = col.reshape(NS, N_CHUNKS, CHUNK)
    row_off = jnp.stack([row, row + N_NODES]).reshape(NC, NS, N_CHUNKS, CHUNK)

    deg2 = _deg_kernel(col5).T
    hs2 = _mmhs_call(x, W_phi, deg2)
    acc = _agg_kernel(hs2.reshape(NC * N_NODES, HALF), row_off, col3)
    return _fin_call(x, W, b.reshape(1, C), acc, deg2)


# speedup vs baseline: 19.9330x; 1.0099x over previous
import dataclasses
import functools

import jax
import jax.numpy as jnp
from jax import lax
from jax.experimental import pallas as pl
from jax.experimental.pallas import tpu as pltpu
from jax.experimental.pallas import tpu_sc as plsc

N_NODES = 10000
N_EDGES = 160000
C = 256
HALF = 128
GAMMA = 0.1
EPS = 0.1

NC = 2
NS = 16
CHUNK = 125
E_PER_SUB = N_EDGES // NS
N_CHUNKS = E_PER_SUB // CHUNK
E_PER_WORKER = N_EDGES // (NC * NS)
N_PAD = 10240
STRIPE = N_PAD // NS

_mesh = plsc.VectorSubcoreMesh(core_axis_name="c", subcore_axis_name="s")


DW = 40
D_ROWS = 125

_cp = pltpu.CompilerParams()
if "needs_layout_passes" in pltpu.CompilerParams.__dataclass_fields__:
    _cp = dataclasses.replace(_cp, needs_layout_passes=False)


@functools.partial(
    pl.kernel,
    mesh=_mesh,
    compiler_params=_cp,
    out_type=jax.ShapeDtypeStruct((NC, N_PAD), jnp.float32),
    scratch_types=[
        pltpu.VMEM((D_ROWS, DW), jnp.int32),
        pltpu.VMEM((N_PAD,), jnp.float32),
        pltpu.VMEM((STRIPE,), jnp.float32),
        pltpu.VMEM((STRIPE,), jnp.float32),
        pltpu.VMEM_SHARED((NS, N_PAD), jnp.float32),
    ],
)
def _deg_kernel(col_hbm, out_hbm, col_v, hist, tmp, accs, stage_sh):
    cid = lax.axis_index("c")
    sid = lax.axis_index("s")
    pltpu.sync_copy(col_hbm.at[cid, sid], col_v)

    @pl.loop(0, N_PAD // 16)
    def _(i):
        hist[pl.ds(i * 16, 16)] = jnp.zeros((16,), jnp.float32)

    one16 = jnp.ones((16,), jnp.float32)
    tail_mask = lax.iota(jnp.int32, 16) >= 8

    @pl.loop(0, D_ROWS)
    def _(r):
        plsc.addupdate_scatter(hist, [col_v[r, pl.ds(0, 16)]], one16)
        plsc.addupdate_scatter(hist, [col_v[r, pl.ds(16, 16)]], one16)
        plsc.addupdate_scatter(hist, [col_v[r, pl.ds(24, 16)]], one16,
                               mask=tail_mask)

    pltpu.sync_copy(hist, stage_sh.at[sid])
    plsc.subcore_barrier()

    @pl.loop(0, STRIPE // 16)
    def _(t):
        accs[pl.ds(t * 16, 16)] = jnp.zeros((16,), jnp.float32)

    @pl.loop(0, NS)
    def _(k):
        pltpu.sync_copy(stage_sh.at[k, pl.ds(sid * STRIPE, STRIPE)], tmp)

        @pl.loop(0, STRIPE // 16)
        def _(t):
            sl = pl.ds(t * 16, 16)
            accs[sl] = accs[sl] + tmp[sl]

    pltpu.sync_copy(accs, out_hbm.at[cid, pl.ds(sid * STRIPE, STRIPE)])


@functools.partial(
    pl.kernel,
    mesh=_mesh,
    out_type=jax.ShapeDtypeStruct((NC, N_PAD, HALF), jnp.float32),
    scratch_types=[
        pltpu.VMEM((N_CHUNKS, CHUNK), jnp.int32),
        pltpu.VMEM((2, CHUNK), jnp.int32),
        pltpu.VMEM((CHUNK, HALF), jnp.float32),
        pltpu.VMEM((CHUNK, HALF), jnp.float32),
        pltpu.VMEM_SHARED((N_PAD, HALF), jnp.float32),
        pltpu.SemaphoreType.DMA,
        pltpu.SemaphoreType.DMA,
        pltpu.SemaphoreType.DMA,
        pltpu.SemaphoreType.DMA,
        pltpu.SemaphoreType.DMA,
        pltpu.SemaphoreType.DMA,
    ],
)
def _agg_kernel(hs_hbm, row_hbm, col_hbm, out_hbm, row_v, col_v, buf_a, buf_b,
                acc_sh, sem_ga, sem_gb, sem_ca, sem_cb, sem_sa, sem_sb):
    cid = lax.axis_index("c")
    sid = lax.axis_index("s")
    hs_c = hs_hbm.at[cid]
    pltpu.sync_copy(row_hbm.at[sid], row_v)

    @pl.when(sid < NS - 1)
    def _():
        pltpu.sync_copy(hs_c.at[pl.ds(sid * STRIPE, STRIPE)],
                        acc_sh.at[pl.ds(sid * STRIPE, STRIPE)])

    @pl.when(sid == NS - 1)
    def _():
        pltpu.sync_copy(
            hs_c.at[pl.ds((NS - 1) * STRIPE, N_NODES - (NS - 1) * STRIPE)],
            acc_sh.at[pl.ds((NS - 1) * STRIPE, N_NODES - (NS - 1) * STRIPE)])

    plsc.subcore_barrier()

    pltpu.async_copy(col_hbm.at[sid, 0], col_v.at[0], sem_ca)
    pltpu.async_copy(col_hbm.at[sid, 1], col_v.at[1], sem_cb)
    pltpu.async_copy(hs_c.at[row_v.at[0]], buf_a, sem_ga)
    pltpu.async_copy(hs_c.at[row_v.at[1]], buf_b, sem_gb)

    @pl.loop(0, N_CHUNKS, step=2)
    def _(j):
        ja = jnp.minimum(j + 2, N_CHUNKS - 1)
        jb = jnp.minimum(j + 3, N_CHUNKS - 1)
        pltpu.make_async_copy(hs_c.at[row_v.at[0]], buf_a, sem_ga).wait()
        pltpu.make_async_copy(col_hbm.at[sid, 0], col_v.at[0], sem_ca).wait()
        pltpu.async_copy(buf_a, acc_sh.at[col_v.at[0]], sem_sa, add=True)
        pltpu.make_async_copy(hs_c.at[row_v.at[0]], buf_b, sem_gb).wait()
        pltpu.make_async_copy(col_hbm.at[sid, 0], col_v.at[1], sem_cb).wait()
        pltpu.async_copy(buf_b, acc_sh.at[col_v.at[1]], sem_sb, add=True)
        pltpu.make_async_copy(buf_a, acc_sh.at[col_v.at[0]], sem_sa).wait()
        pltpu.async_copy(col_hbm.at[sid, ja], col_v.at[0], sem_ca)
        pltpu.async_copy(hs_c.at[row_v.at[ja]], buf_a, sem_ga)
        pltpu.make_async_copy(buf_b, acc_sh.at[col_v.at[1]], sem_sb).wait()
        pltpu.async_copy(col_hbm.at[sid, jb], col_v.at[1], sem_cb)
        pltpu.async_copy(hs_c.at[row_v.at[jb]], buf_b, sem_gb)

    pltpu.make_async_copy(hs_c.at[row_v.at[0]], buf_a, sem_ga).wait()
    pltpu.make_async_copy(hs_c.at[row_v.at[0]], buf_b, sem_gb).wait()
    pltpu.make_async_copy(col_hbm.at[sid, 0], col_v.at[0], sem_ca).wait()
    pltpu.make_async_copy(col_hbm.at[sid, 0], col_v.at[1], sem_cb).wait()

    plsc.subcore_barrier()
    pltpu.sync_copy(
        acc_sh.at[pl.ds(sid * STRIPE, STRIPE)],
        out_hbm.at[cid, pl.ds(sid * STRIPE, STRIPE)],
    )


_BLK = 1000


def _mmhs_body(x_ref, wphi_ref, deg_ref, hs_ref):
    d = deg_ref[:, 0:1] + deg_ref[:, 1:2] + 1.0
    dinv = lax.rsqrt(d)
    h = jnp.dot(x_ref[...], wphi_ref[...], preferred_element_type=jnp.float32)
    hs_ref[0] = dinv * h[:, :HALF]
    hs_ref[1] = dinv * h[:, HALF:]


def _mmhs_call(x, wphi, deg2):
    return pl.pallas_call(
        _mmhs_body,
        grid=(N_NODES // _BLK,),
        in_specs=[
            pl.BlockSpec((_BLK, C), lambda i: (i, 0)),
            pl.BlockSpec((C, C), lambda i: (0, 0)),
            pl.BlockSpec((_BLK, NC), lambda i: (i, 0)),
        ],
        out_specs=pl.BlockSpec((NC, _BLK, HALF), lambda i: (0, i, 0)),
        out_shape=jax.ShapeDtypeStruct((NC, N_NODES, HALF), jnp.float32),
    )(x, wphi, deg2)


def _fin_body(x_ref, at_ref, b_ref, acc_ref, deg_ref, o_ref):
    xb = x_ref[...]
    d = deg_ref[:, 0:1] + deg_ref[:, 1:2] + 1.0
    dinv = lax.rsqrt(d)
    z = jnp.dot(xb, at_ref[...], preferred_element_type=jnp.float32) + b_ref[...]
    accf = jnp.concatenate([acc_ref[0], acc_ref[1]], axis=-1)
    o_ref[...] = xb + EPS * jnp.tanh(z + dinv * accf)


def _fin_call(x, w, b2, acc, deg2):
    return pl.pallas_call(
        _fin_body,
        grid=(N_NODES // _BLK,),
        in_specs=[
            pl.BlockSpec((_BLK, C), lambda i: (i, 0)),
            pl.BlockSpec((C, C), lambda i: (0, 0)),
            pl.BlockSpec((1, C), lambda i: (0, 0)),
            pl.BlockSpec((NC, _BLK, HALF), lambda i: (0, i, 0)),
            pl.BlockSpec((_BLK, NC), lambda i: (i, 0)),
        ],
        out_specs=pl.BlockSpec((_BLK, C), lambda i: (i, 0)),
        out_shape=jax.ShapeDtypeStruct((N_NODES, C), jnp.float32),
    )(x, w, b2, acc, deg2)


def kernel(x, edge_index, W, b, W_phi):
    ei = edge_index.astype(jnp.int32)
    row = ei[0]
    col = ei[1]
    col3 = col.reshape(NS, N_CHUNKS, CHUNK)
    col_d = col.reshape(NC, NS, D_ROWS, DW)
    row3 = row.reshape(NS, N_CHUNKS, CHUNK)
    a_t = W.T - W - GAMMA * jnp.eye(C, dtype=W.dtype)

    deg2 = _deg_kernel(col_d).T
    hs2 = _mmhs_call(x, W_phi, deg2)
    acc = _agg_kernel(hs2, row3, col3)
    return _fin_call(x, a_t, b.reshape(1, C), acc, deg2)
